# Initial kernel scaffold; baseline (speedup 1.0000x reference)
import jax
import jax.numpy as jnp
from jax.experimental import pallas as pl


def kernel(user_emb_weight, item_emb_weight, edge_index):
    raise NotImplementedError("write your pallas kernel here")


# speedup vs baseline: 7.1003x; 7.1003x over previous
import functools

import jax
import jax.numpy as jnp
from jax import lax
from jax.experimental import pallas as pl
from jax.experimental.pallas import tpu as pltpu
from jax.experimental.pallas import tpu_sc as plsc

N_USERS = 25000
N_NODES = 50000
D = 64
H = 32
E = 800000
CH = 128
N_TILES = 16
PTE = 50176
EP = PTE * N_TILES
NCH = PTE // CH
RA = 50176
PTR = RA // N_TILES
ZR = 392
TRASH = 50000

_mesh = plsc.VectorSubcoreMesh(core_axis_name="c", subcore_axis_name="s")


@functools.partial(
    pl.kernel,
    mesh=_mesh,
    compiler_params=pltpu.CompilerParams(use_tc_tiling_on_sc=False),
    out_type=pltpu.HBM((2 * RA, H), jnp.float32),
    scratch_types=[
        pltpu.VMEM_SHARED((RA, H), jnp.float32),
        pltpu.VMEM((ZR, H), jnp.float32),
        pltpu.VMEM((CH,), jnp.int32),
        pltpu.VMEM((CH,), jnp.int32),
        pltpu.VMEM((CH, H), jnp.float32),
        pltpu.SemaphoreType.DMA,
    ],
)
def _propagate(y2, rowp, col2p, zrows, out, acc, zbuf, rowv, colv, rowsv,
               sem):
    c = lax.axis_index("c")
    s = lax.axis_index("s")

    pltpu.sync_copy(zrows, zbuf)
    for i in range(PTR // ZR):
        pltpu.sync_copy(zbuf, acc.at[pl.ds(s * PTR + i * ZR, ZR), :])
    plsc.subcore_barrier()

    def chunk(j, carry):
        ebase = s * PTE + j * CH
        pltpu.sync_copy(rowp.at[pl.ds(ebase, CH)], rowv)
        pltpu.sync_copy(col2p.at[pl.ds(c * EP + ebase, CH)], colv)
        pltpu.async_copy(y2.at[colv], rowsv, sem).wait()
        pltpu.sync_copy(rowsv, acc.at[rowv], add=True)
        return carry

    lax.fori_loop(0, NCH, chunk, 0)
    plsc.subcore_barrier()

    for i in range(PTR // ZR):
        pltpu.sync_copy(acc.at[pl.ds(s * PTR + i * ZR, ZR), :], zbuf)
        pltpu.sync_copy(
            zbuf, out.at[pl.ds(c * RA + s * PTR + i * ZR, ZR), :])


def kernel(user_emb_weight, item_emb_weight, edge_index):
    row = edge_index[0].astype(jnp.int32)
    col = edge_index[1].astype(jnp.int32)

    deg = jnp.bincount(row, length=N_NODES).astype(jnp.float32) + 1e-07
    dis = jnp.power(deg, -0.5)

    rowp = jnp.concatenate([row, jnp.full((EP - E,), TRASH, jnp.int32)])
    colp = jnp.concatenate([col, jnp.zeros((EP - E,), jnp.int32)])
    col2p = jnp.concatenate([colp, colp + N_NODES])
    zrows = jnp.zeros((ZR, H), jnp.float32)

    x = jnp.concatenate([user_emb_weight, item_emb_weight], axis=0)
    acc = x
    y = x * dis[:, None]
    for _ in range(3):
        y2 = jnp.concatenate([y[:, :H], y[:, H:]], axis=0)
        o = _propagate(y2, rowp, col2p, zrows)
        o = jax.device_put(o, jax.memory.Space.Device)
        seg = jnp.concatenate(
            [o[:N_NODES, :], o[RA:RA + N_NODES, :]], axis=1)
        x = seg * dis[:, None]
        acc = acc + x
        y = x * dis[:, None]

    mean = acc * 0.25
    return mean[:N_USERS], mean[N_USERS:]


# speedup vs baseline: 9.6839x; 1.3639x over previous
import functools

import jax
import jax.numpy as jnp
from jax import lax
from jax.experimental import pallas as pl
from jax.experimental.pallas import tpu as pltpu
from jax.experimental.pallas import tpu_sc as plsc

N_USERS = 25000
N_NODES = 50000
D = 64
H = 32
E = 800000
CH = 128
N_TILES = 16
PTE = 50176
EP = PTE * N_TILES
NCH = PTE // CH
RA = 50176
PTR = RA // N_TILES
ZR = 196
NSLOT = 4
TRASH = 50000

_mesh = plsc.VectorSubcoreMesh(core_axis_name="c", subcore_axis_name="s")


@functools.partial(
    pl.kernel,
    mesh=_mesh,
    compiler_params=pltpu.CompilerParams(use_tc_tiling_on_sc=False),
    out_type=pltpu.HBM((2 * RA, H), jnp.float32),
    scratch_types=[
        pltpu.VMEM_SHARED((RA, H), jnp.float32),
        pltpu.VMEM((ZR, H), jnp.float32),
    ] + [pltpu.VMEM((CH,), jnp.int32) for _ in range(NSLOT)]
      + [pltpu.VMEM((CH,), jnp.int32) for _ in range(NSLOT)]
      + [pltpu.VMEM((CH, H), jnp.float32) for _ in range(NSLOT)]
      + [pltpu.SemaphoreType.DMA for _ in range(NSLOT)],
)
def _propagate(y2, rowp, col2p, zrows, out, acc, zbuf, *slots):
    rowvs = slots[0:NSLOT]
    colvs = slots[NSLOT:2 * NSLOT]
    rowsvs = slots[2 * NSLOT:3 * NSLOT]
    sems = slots[3 * NSLOT:4 * NSLOT]
    c = lax.axis_index("c")
    s = lax.axis_index("s")

    pltpu.sync_copy(zrows, zbuf)
    for i in range(PTR // ZR):
        pltpu.sync_copy(zbuf, acc.at[pl.ds(s * PTR + i * ZR, ZR), :])
    plsc.subcore_barrier()

    def chunk(j, carry):
        copies = []
        for k in range(NSLOT):
            ebase = s * PTE + (j * NSLOT + k) * CH
            pltpu.sync_copy(rowp.at[pl.ds(ebase, CH)], rowvs[k])
            pltpu.sync_copy(col2p.at[pl.ds(c * EP + ebase, CH)], colvs[k])
            copies.append(
                pltpu.async_copy(y2.at[colvs[k]], rowsvs[k], sems[k]))
        for k in range(NSLOT):
            copies[k].wait()
            pltpu.sync_copy(rowsvs[k], acc.at[rowvs[k]], add=True)
        return carry

    lax.fori_loop(0, NCH // NSLOT, chunk, 0)
    plsc.subcore_barrier()

    for i in range(PTR // ZR):
        pltpu.sync_copy(acc.at[pl.ds(s * PTR + i * ZR, ZR), :], zbuf)
        pltpu.sync_copy(
            zbuf, out.at[pl.ds(c * RA + s * PTR + i * ZR, ZR), :])


def kernel(user_emb_weight, item_emb_weight, edge_index):
    row = edge_index[0].astype(jnp.int32)
    col = edge_index[1].astype(jnp.int32)

    deg = jnp.bincount(row, length=N_NODES).astype(jnp.float32) + 1e-07
    dis = jnp.power(deg, -0.5)

    rowp = jnp.concatenate([row, jnp.full((EP - E,), TRASH, jnp.int32)])
    colp = jnp.concatenate([col, jnp.zeros((EP - E,), jnp.int32)])
    col2p = jnp.concatenate([colp, colp + N_NODES])
    zrows = jnp.zeros((ZR, H), jnp.float32)

    x = jnp.concatenate([user_emb_weight, item_emb_weight], axis=0)
    acc = x
    y = x * dis[:, None]
    for _ in range(3):
        y2 = jnp.concatenate([y[:, :H], y[:, H:]], axis=0)
        o = _propagate(y2, rowp, col2p, zrows)
        o = jax.device_put(o, jax.memory.Space.Device)
        seg = jnp.concatenate(
            [o[:N_NODES, :], o[RA:RA + N_NODES, :]], axis=1)
        x = seg * dis[:, None]
        acc = acc + x
        y = x * dis[:, None]

    mean = acc * 0.25
    return mean[:N_USERS], mean[N_USERS:]
